# baseline (speedup 1.0000x reference)
import functools

import jax
import jax.numpy as jnp
from jax import lax
from jax.experimental import pallas as pl
from jax.experimental.pallas import tpu as pltpu
from jax.experimental.pallas import tpu_sc as plsc

B, X, L = 8, 768, 576
H, D = 8, 96
K, C = 8192, 256
SCALE = D ** (-0.5)


KB1 = 1024


DP = 128


def _proj_body(codes_ref, wk_ref, wv_ref, k_ref, v_ref):
    cds = codes_ref[...]
    kk = jnp.dot(cds, wk_ref[...], preferred_element_type=jnp.float32)
    for h in range(H):
        k_ref[h] = kk[:, h * D:(h + 1) * D]
    v_ref[...] = jnp.dot(cds, wv_ref[...], preferred_element_type=jnp.float32)


def _project(codes, wk_flat, wv_pad):
    return pl.pallas_call(
        _proj_body,
        grid=(K // KB1,),
        in_specs=[
            pl.BlockSpec((KB1, C), lambda i: (i, 0)),
            pl.BlockSpec((C, H * D), lambda i: (0, 0)),
            pl.BlockSpec((C, H * DP), lambda i: (0, 0)),
        ],
        out_specs=[
            pl.BlockSpec((H, KB1, D), lambda i: (0, i, 0)),
            pl.BlockSpec((KB1, H * DP), lambda i: (i, 0)),
        ],
        out_shape=[
            jax.ShapeDtypeStruct((H, K, D), jnp.float32),
            jax.ShapeDtypeStruct((K, H * DP), jnp.float32),
        ],
    )(codes, wk_flat, wv_pad)


KM = 8192
KT = K // KM


S = H * B * KT


def _hbk(s):
    h = s // (B * KT)
    r = s % (B * KT)
    return h, r // KT, r % KT


def _argmax_body(x_ref, k_ref, idx_ref, gidx_ref):
    s = pl.program_id(0)
    h = s // B
    q = x_ref[0] * SCALE
    kk = k_ref[0]
    lt = lax.dot_general(kk, q, (((1,), (0,)), ((), ())),
                         preferred_element_type=jnp.float32)
    tile_max = jnp.max(lt, axis=0, keepdims=True)
    eqb = jnp.where(lt == tile_max, 1.0, 0.0).astype(jnp.bfloat16)
    ii = lax.broadcasted_iota(jnp.int32, (3, KM), 1)
    rr = lax.broadcasted_iota(jnp.int32, (3, KM), 0)
    iot3 = jnp.where(rr == 0, ii // 256,
                     jnp.where(rr == 1, (ii // 16) % 16, ii % 16))
    iot3 = iot3.astype(jnp.bfloat16)
    r3 = lax.dot_general(iot3, eqb, (((1,), (0,)), ((), ())),
                         preferred_element_type=jnp.float32)
    idx_f = r3[0:1] * 256.0 + r3[1:2] * 16.0 + r3[2:3]
    idx_i = idx_f.astype(jnp.int32)
    idx_ref[...] = idx_i.reshape(1, 1, 1, L)
    gidx_ref[...] = (idx_i * H + h).reshape(1, 1, 1, L)


def _argmax(x, k3):

    def xmap(s):
        h, b, _ = _hbk(s)
        return (b, h, 0)

    def kmap(s):
        h, _, kt = _hbk(s)
        return (h, kt, 0)

    def omap(s):
        h, b, _ = _hbk(s)
        return (b, h, 0, 0)

    return pl.pallas_call(
        _argmax_body,
        grid=(S,),
        in_specs=[
            pl.BlockSpec((1, D, L), xmap),
            pl.BlockSpec((1, KM, D), kmap),
        ],
        out_specs=[
            pl.BlockSpec((1, 1, 1, L), omap),
            pl.BlockSpec((1, 1, 1, L), omap),
        ],
        out_shape=[
            jax.ShapeDtypeStruct((B, H, 1, L), jnp.int32),
            jax.ShapeDtypeStruct((B, H, 1, L), jnp.int32),
        ],
        compiler_params=pltpu.CompilerParams(
            dimension_semantics=("arbitrary",),
        ),
    )(x, k3)


NW = 32
ROWS = B * H * L
RPW = ROWS // NW
CHUNK = 128
NCH = RPW // CHUNK


@functools.cache
def _make_sc_gather():
    mesh = plsc.VectorSubcoreMesh(core_axis_name="c", subcore_axis_name="s")

    @functools.partial(
        pl.kernel,
        mesh=mesh,
        out_type=jax.ShapeDtypeStruct((ROWS, DP), jnp.float32),
        scratch_types=[
            pltpu.VMEM((NCH, CHUNK), jnp.int32),
            pltpu.VMEM((2, CHUNK, DP), jnp.float32),
            pltpu.SemaphoreType.DMA,
            pltpu.SemaphoreType.DMA,
            pltpu.SemaphoreType.DMA,
        ],
        compiler_params=pltpu.CompilerParams(use_tc_tiling_on_sc=True),
    )
    def sc_gather(table_hbm, gidx_hbm, out_hbm, idx_v, buf, gsem, wsem0,
                  wsem1):
        wid = lax.axis_index("s") * 2 + lax.axis_index("c")
        base = wid * RPW
        pltpu.sync_copy(gidx_hbm.at[wid], idx_v)
        wsems = (wsem0, wsem1)

        def gather(j):
            return pltpu.async_copy(table_hbm.at[idx_v.at[j]], buf.at[j % 2],
                                    gsem)

        def writeback(j):
            return pltpu.async_copy(
                buf.at[j % 2], out_hbm.at[pl.ds(base + j * CHUNK, CHUNK)],
                wsems[j % 2])

        g = gather(0)
        wbs = [None, None]
        for j in range(NCH):
            g.wait()
            if j + 1 < NCH:
                nxt = (j + 1) % 2
                if wbs[nxt] is not None:
                    wbs[nxt].wait()
                    wbs[nxt] = None
                g = gather(j + 1)
            wbs[j % 2] = writeback(j)
        for wb in wbs:
            if wb is not None:
                wb.wait()

    return sc_gather


def kernel(x, codes, Wk, Wv):
    wv_pad = jnp.pad(Wv, ((0, 0), (0, 0), (0, DP - D))).reshape(C, H * DP)
    k3, v_flat = _project(codes, Wk.reshape(C, H * D), wv_pad)
    idx4, gidx4 = _argmax(x, k3)
    rows = _make_sc_gather()(v_flat.reshape(K * H, DP),
                             gidx4.reshape(NW, NCH, CHUNK))
    out = (rows.reshape(B, H, L, DP)[..., :D]
           .transpose(0, 1, 3, 2).reshape(B, H * D, L))
    return out, idx4.reshape(B, H, L)


# speedup vs baseline: 1.0728x; 1.0728x over previous
import functools

import jax
import jax.numpy as jnp
from jax import lax
from jax.experimental import pallas as pl
from jax.experimental.pallas import tpu as pltpu
from jax.experimental.pallas import tpu_sc as plsc

B, X, L = 8, 768, 576
H, D = 8, 96
K, C = 8192, 256
SCALE = D ** (-0.5)


KB1 = 1024


DP = 128


def _proj_body(codes_ref, wk_ref, wv_ref, k_ref, v_ref):
    cds = codes_ref[...]
    kk = jnp.dot(cds, wk_ref[...], preferred_element_type=jnp.float32)
    for h in range(H):
        k_ref[h] = kk[:, h * D:(h + 1) * D]
    for h in range(H):
        v_ref[h] = jnp.dot(cds, wv_ref[h], preferred_element_type=jnp.float32)


def _project(codes, wk_flat, wv_t):
    return pl.pallas_call(
        _proj_body,
        grid=(K // KB1,),
        in_specs=[
            pl.BlockSpec((KB1, C), lambda i: (i, 0)),
            pl.BlockSpec((C, H * D), lambda i: (0, 0)),
            pl.BlockSpec((H, C, DP), lambda i: (0, 0, 0)),
        ],
        out_specs=[
            pl.BlockSpec((H, KB1, D), lambda i: (0, i, 0)),
            pl.BlockSpec((H, KB1, DP), lambda i: (0, i, 0)),
        ],
        out_shape=[
            jax.ShapeDtypeStruct((H, K, D), jnp.float32),
            jax.ShapeDtypeStruct((H, K, DP), jnp.float32),
        ],
    )(codes, wk_flat, wv_t)


KM = 8192
KT = K // KM


S = H * B * KT


def _hbk(s):
    h = s // (B * KT)
    r = s % (B * KT)
    return h, r // KT, r % KT


def _argmax_body(x_ref, k_ref, idx_ref, gidx_ref):
    s = pl.program_id(0)
    h = s // B
    q = x_ref[0] * SCALE
    kk = k_ref[0]
    lt = lax.dot_general(kk, q, (((1,), (0,)), ((), ())),
                         preferred_element_type=jnp.float32)
    tile_max = jnp.max(lt, axis=0, keepdims=True)
    eqb = jnp.where(lt == tile_max, 1.0, 0.0).astype(jnp.bfloat16)
    ii = lax.broadcasted_iota(jnp.int32, (3, KM), 1)
    rr = lax.broadcasted_iota(jnp.int32, (3, KM), 0)
    iot3 = jnp.where(rr == 0, ii // 256,
                     jnp.where(rr == 1, (ii // 16) % 16, ii % 16))
    iot3 = iot3.astype(jnp.bfloat16)
    r3 = lax.dot_general(iot3, eqb, (((1,), (0,)), ((), ())),
                         preferred_element_type=jnp.float32)
    idx_f = r3[0:1] * 256.0 + r3[1:2] * 16.0 + r3[2:3]
    idx_i = idx_f.astype(jnp.int32)
    idx_ref[...] = idx_i.reshape(1, 1, 1, L)
    gidx_ref[...] = (idx_i + h * K).reshape(1, 1, 1, L)


def _argmax(x, k3):

    def xmap(s):
        h, b, _ = _hbk(s)
        return (b, h, 0)

    def kmap(s):
        h, _, kt = _hbk(s)
        return (h, kt, 0)

    def omap(s):
        h, b, _ = _hbk(s)
        return (b, h, 0, 0)

    return pl.pallas_call(
        _argmax_body,
        grid=(S,),
        in_specs=[
            pl.BlockSpec((1, D, L), xmap),
            pl.BlockSpec((1, KM, D), kmap),
        ],
        out_specs=[
            pl.BlockSpec((1, 1, 1, L), omap),
            pl.BlockSpec((1, 1, 1, L), omap),
        ],
        out_shape=[
            jax.ShapeDtypeStruct((B, H, 1, L), jnp.int32),
            jax.ShapeDtypeStruct((B, H, 1, L), jnp.int32),
        ],
        compiler_params=pltpu.CompilerParams(
            dimension_semantics=("arbitrary",),
        ),
    )(x, k3)


NW = 32
ROWS = B * H * L
RPW = ROWS // NW
CHUNK = 128
NCH = RPW // CHUNK


@functools.cache
def _make_sc_gather():
    mesh = plsc.VectorSubcoreMesh(core_axis_name="c", subcore_axis_name="s")

    @functools.partial(
        pl.kernel,
        mesh=mesh,
        out_type=jax.ShapeDtypeStruct((ROWS, DP), jnp.float32),
        scratch_types=[
            pltpu.VMEM((NCH, CHUNK), jnp.int32),
            pltpu.VMEM((2, CHUNK, DP), jnp.float32),
            pltpu.SemaphoreType.DMA,
            pltpu.SemaphoreType.DMA,
            pltpu.SemaphoreType.DMA,
        ],
        compiler_params=pltpu.CompilerParams(use_tc_tiling_on_sc=True),
    )
    def sc_gather(table_hbm, gidx_hbm, out_hbm, idx_v, buf, gsem, wsem0,
                  wsem1):
        wid = lax.axis_index("s") * 2 + lax.axis_index("c")
        base = wid * RPW
        pltpu.sync_copy(gidx_hbm.at[wid], idx_v)
        wsems = (wsem0, wsem1)

        def gather(j):
            return pltpu.async_copy(table_hbm.at[idx_v.at[j]], buf.at[j % 2],
                                    gsem)

        def writeback(j):
            return pltpu.async_copy(
                buf.at[j % 2], out_hbm.at[pl.ds(base + j * CHUNK, CHUNK)],
                wsems[j % 2])

        g = gather(0)
        wbs = [None, None]
        for j in range(NCH):
            g.wait()
            if j + 1 < NCH:
                nxt = (j + 1) % 2
                if wbs[nxt] is not None:
                    wbs[nxt].wait()
                    wbs[nxt] = None
                g = gather(j + 1)
            wbs[j % 2] = writeback(j)
        for wb in wbs:
            if wb is not None:
                wb.wait()

    return sc_gather


def kernel(x, codes, Wk, Wv):
    wv_t = jnp.pad(Wv, ((0, 0), (0, 0), (0, DP - D))).transpose(1, 0, 2)
    k3, v3 = _project(codes, Wk.reshape(C, H * D), wv_t)
    idx4, gidx4 = _argmax(x, k3)
    rows = _make_sc_gather()(v3.reshape(H * K, DP),
                             gidx4.reshape(NW, NCH, CHUNK))
    out = (rows.reshape(B, H, L, DP)[..., :D]
           .transpose(0, 1, 3, 2).reshape(B, H * D, L))
    return out, idx4.reshape(B, H, L)
